# baseline (speedup 1.0000x reference)
import functools

import jax
import jax.numpy as jnp
from jax import lax
from jax.experimental import pallas as pl
from jax.experimental.pallas import tpu as pltpu
from jax.experimental.pallas import tpu_sc as plsc

N = 10000
NPAD = 10240
E = 320000
D = 128

NC = 2
NS = 16
NW = NC * NS

C = 40
EPW = E // NW
NCH = EPW // C
EPT = E // NS
NCHD = EPT // C
NPW = NPAD // NW

_mesh = plsc.VectorSubcoreMesh(core_axis_name="c", subcore_axis_name="s")
_sc_params = pltpu.CompilerParams(needs_layout_passes=False)


def _prep_body(rowp_hbm, x_hbm, dis_hbm, y_hbm,
               deg_sh, rowv, onesv, zdeg, dgv, disv, xv, dsem):
    c = lax.axis_index("c")
    s = lax.axis_index("s")
    for j in range(C // 16):
        onesv[pl.ds(16 * j, 16)] = jnp.ones((16,), jnp.float32)
    for j in range(40):
        zdeg[pl.ds(16 * j, 16)] = jnp.zeros((16,), jnp.float32)
    pltpu.sync_copy(zdeg, deg_sh.at[pl.ds(640 * s, 640)])
    plsc.subcore_barrier()
    pltpu.sync_copy(rowp_hbm.at[s], rowv)
    for grp in range(NCHD // 25):
        descs = [
            pltpu.async_copy(onesv, deg_sh.at[rowv.at[25 * grp + k]],
                             dsem, add=True)
            for k in range(25)
        ]
        for d in descs:
            d.wait()
    plsc.subcore_barrier()
    nb = 5120 * c + 320 * s
    pltpu.sync_copy(deg_sh.at[pl.ds(nb, NPW)], dgv)
    for j in range(NPW // 16):
        dg = dgv[pl.ds(16 * j, 16)]
        dgc = jnp.maximum(dg, 1.0)
        u = lax.bitcast_convert_type(dgc, jnp.int32)
        u = jnp.int32(0x5F3759DF) - (u >> 1)
        h = lax.bitcast_convert_type(u, jnp.float32)
        for _ in range(3):
            h = h * (1.5 - 0.5 * dgc * h * h)
        disv[pl.ds(16 * j, 16)] = jnp.where(dg > 0.0, h, 0.0)
    pltpu.sync_copy(disv, dis_hbm.at[pl.ds(nb, NPW)])
    pltpu.sync_copy(x_hbm.at[pl.ds(nb, NPW)], xv)

    def nbody(n, carry):
        idxn = jnp.zeros((16,), jnp.int32) + n
        s16 = plsc.load_gather(disv, [idxn])
        for j in range(D // 16):
            xv[n, pl.ds(16 * j, 16)] = xv[n, pl.ds(16 * j, 16)] * s16
        return carry

    lax.fori_loop(0, NPW, nbody, 0)
    pltpu.sync_copy(xv, y_hbm.at[pl.ds(nb, NPW)])


@functools.partial(
    pl.kernel,
    out_type=(
        jax.ShapeDtypeStruct((NPAD,), jnp.float32),
        jax.ShapeDtypeStruct((NPAD, D), jnp.float32),
    ),
    mesh=_mesh,
    scratch_types=[
        pltpu.VMEM_SHARED((NPAD,), jnp.float32),
        pltpu.VMEM((NCHD, C), jnp.int32),
        pltpu.VMEM((C,), jnp.float32),
        pltpu.VMEM((640,), jnp.float32),
        pltpu.VMEM((NPW,), jnp.float32),
        pltpu.VMEM((NPW,), jnp.float32),
        pltpu.VMEM((NPW, D), jnp.float32),
        pltpu.SemaphoreType.DMA,
    ],
    compiler_params=_sc_params,
)
def _prep(*args):
    _prep_body(*args)


BLK = 25
NBLK = NCH // BLK


def _agg_body(row3_hbm, col3_hbm, y_hbm, z2_hbm,
              acc_sh, rowv, colv, gbuf, zsrc, gsem, ssem, zsem):
    c = lax.axis_index("c")
    s = lax.axis_index("s")
    w = s * NC + c

    def zbody(k, carry):
        for j in range(D // 16):
            zsrc[k, pl.ds(16 * j, 16)] = jnp.zeros((16,), jnp.float32)
        return carry

    lax.fori_loop(0, 40, zbody, 0)
    zd = [
        pltpu.async_copy(zsrc, acc_sh.at[pl.ds(640 * s + 40 * k, 40)], zsem)
        for k in range(16)
    ]
    for d in zd:
        d.wait()
    plsc.subcore_barrier()
    for blk in range(NBLK):
        pltpu.sync_copy(row3_hbm.at[w, blk], rowv)
        pltpu.sync_copy(col3_hbm.at[w, blk], colv)
        g = [None] * BLK
        sc = [None] * BLK
        for p in range(4):
            g[p] = pltpu.async_copy(y_hbm.at[colv.at[p]], gbuf.at[p], gsem)
        for t in range(BLK):
            if t >= 2:
                sc[t - 2].wait()
            if t + 4 < BLK:
                g[t + 4] = pltpu.async_copy(
                    y_hbm.at[colv.at[t + 4]], gbuf.at[(t + 4) % 6], gsem)
            g[t].wait()
            sc[t] = pltpu.async_copy(
                gbuf.at[t % 6], acc_sh.at[rowv.at[t]], ssem, add=True)
        sc[BLK - 2].wait()
        sc[BLK - 1].wait()
    plsc.subcore_barrier()
    pltpu.sync_copy(acc_sh.at[pl.ds(640 * s, 640)],
                    z2_hbm.at[c, pl.ds(640 * s, 640)])


@functools.partial(
    pl.kernel,
    out_type=jax.ShapeDtypeStruct((NC, NPAD, D), jnp.float32),
    mesh=_mesh,
    scratch_types=[
        pltpu.VMEM_SHARED((NPAD, D), jnp.float32),
        pltpu.VMEM((BLK, C), jnp.int32),
        pltpu.VMEM((BLK, C), jnp.int32),
        pltpu.VMEM((6, C, D), jnp.float32),
        pltpu.VMEM((40, D), jnp.float32),
        pltpu.SemaphoreType.DMA,
        pltpu.SemaphoreType.DMA,
        pltpu.SemaphoreType.DMA,
    ],
    compiler_params=_sc_params,
)
def _agg(*args):
    _agg_body(*args)


def _mm_body(z2_ref, dis_ref, w_ref, b_ref, o_ref):
    z = z2_ref[0] + z2_ref[1]
    zd = z * dis_ref[...]
    r = lax.dot_general(
        zd, w_ref[...], (((1,), (1,)), ((), ())),
        preferred_element_type=jnp.float32,
        precision=lax.Precision.HIGHEST,
    )
    o_ref[...] = r + b_ref[...]


def _mm(z2, dis_col, W, b2):
    blk = 256
    grid = NPAD // blk
    return pl.pallas_call(
        _mm_body,
        grid=(grid,),
        in_specs=[
            pl.BlockSpec((NC, blk, D), lambda i: (0, i, 0)),
            pl.BlockSpec((blk, 1), lambda i: (i, 0)),
            pl.BlockSpec((D, D), lambda i: (0, 0)),
            pl.BlockSpec((1, D), lambda i: (0, 0)),
        ],
        out_specs=pl.BlockSpec((blk, D), lambda i: (i, 0)),
        out_shape=jax.ShapeDtypeStruct((N, D), jnp.float32),
    )(z2, dis_col, W, b2)


@jax.jit
def kernel(x, edge_index, W, b):
    row = edge_index[0].astype(jnp.int32)
    col = edge_index[1].astype(jnp.int32)
    rowp = row.reshape(NS, NCHD, C)
    row3 = row.reshape(NW, NBLK, BLK, C)
    col3 = col.reshape(NW, NBLK, BLK, C)
    x_pad = jnp.pad(x, ((0, NPAD - N), (0, 0)))
    dis, y = _prep(rowp, x_pad)
    z2 = _agg(row3, col3, y)
    return _mm(z2, dis.reshape(NPAD, 1), W, b.reshape(1, D))


# speedup vs baseline: 1.1277x; 1.1277x over previous
import functools

import jax
import jax.numpy as jnp
from jax import lax
from jax.experimental import pallas as pl
from jax.experimental.pallas import tpu as pltpu
from jax.experimental.pallas import tpu_sc as plsc

N = 10000
NPAD = 10240
E = 320000
D = 128

NC = 2
NS = 16
NW = NC * NS

C = 80
EPW = E // NW
NCH = EPW // C
EPT = E // NS
NCHD = EPT // C
NPW = NPAD // NW

_mesh = plsc.VectorSubcoreMesh(core_axis_name="c", subcore_axis_name="s")
_sc_params = pltpu.CompilerParams(needs_layout_passes=False)


def _prep_body(rowp_hbm, x_hbm, dis_hbm, y_hbm,
               deg_sh, rowv, onesv, zdeg, dgv, disv, xv, dsem, xsem):
    c = lax.axis_index("c")
    s = lax.axis_index("s")
    for j in range(C // 16):
        onesv[pl.ds(16 * j, 16)] = jnp.ones((16,), jnp.float32)
    for j in range(40):
        zdeg[pl.ds(16 * j, 16)] = jnp.zeros((16,), jnp.float32)
    pltpu.sync_copy(zdeg, deg_sh.at[pl.ds(640 * s, 640)])
    nb = 5120 * c + 320 * s
    xd = pltpu.async_copy(x_hbm.at[pl.ds(nb, NPW)], xv, xsem)
    plsc.subcore_barrier()
    pltpu.sync_copy(rowp_hbm.at[s], rowv)
    descs = [None] * NCHD
    for k in range(NCHD):
        if k >= 25:
            descs[k - 25].wait()
        descs[k] = pltpu.async_copy(onesv, deg_sh.at[rowv.at[k]],
                                    dsem, add=True)
    for k in range(NCHD - 25, NCHD):
        descs[k].wait()
    plsc.subcore_barrier()
    pltpu.sync_copy(deg_sh.at[pl.ds(nb, NPW)], dgv)
    for j in range(NPW // 16):
        dg = dgv[pl.ds(16 * j, 16)]
        dgc = jnp.maximum(dg, 1.0)
        u = lax.bitcast_convert_type(dgc, jnp.int32)
        u = jnp.int32(0x5F3759DF) - (u >> 1)
        h = lax.bitcast_convert_type(u, jnp.float32)
        for _ in range(3):
            h = h * (1.5 - 0.5 * dgc * h * h)
        disv[pl.ds(16 * j, 16)] = jnp.where(dg > 0.0, h, 0.0)
    pltpu.sync_copy(disv, dis_hbm.at[pl.ds(nb, NPW)])
    xd.wait()

    def nbody(n, carry):
        idxn = jnp.zeros((16,), jnp.int32) + n
        s16 = plsc.load_gather(disv, [idxn])
        for j in range(D // 16):
            xv[n, pl.ds(16 * j, 16)] = xv[n, pl.ds(16 * j, 16)] * s16
        return carry

    lax.fori_loop(0, NPW, nbody, 0)
    pltpu.sync_copy(xv, y_hbm.at[pl.ds(nb, NPW)])


@functools.partial(
    pl.kernel,
    out_type=(
        jax.ShapeDtypeStruct((NPAD,), jnp.float32),
        jax.ShapeDtypeStruct((NPAD, D), jnp.float32),
    ),
    mesh=_mesh,
    scratch_types=[
        pltpu.VMEM_SHARED((NPAD,), jnp.float32),
        pltpu.VMEM((NCHD, C), jnp.int32),
        pltpu.VMEM((C,), jnp.float32),
        pltpu.VMEM((640,), jnp.float32),
        pltpu.VMEM((NPW,), jnp.float32),
        pltpu.VMEM((NPW,), jnp.float32),
        pltpu.VMEM((NPW, D), jnp.float32),
        pltpu.SemaphoreType.DMA,
        pltpu.SemaphoreType.DMA,
    ],
    compiler_params=_sc_params,
)
def _prep(*args):
    _prep_body(*args)


BLK = 25
NBLK = NCH // BLK


def _agg_body(row3_hbm, col3_hbm, y_hbm, z2_hbm,
              acc_sh, rowv, colv, gbuf, gsem, ssem, zsem, isem):
    c = lax.axis_index("c")
    s = lax.axis_index("s")
    w = s * NC + c

    def zbody(k, carry):
        for j in range(D // 16):
            gbuf[0, k, pl.ds(16 * j, 16)] = jnp.zeros((16,), jnp.float32)
        return carry

    lax.fori_loop(0, C, zbody, 0)
    zd = [
        pltpu.async_copy(gbuf.at[0], acc_sh.at[pl.ds(640 * s + C * k, C)], zsem)
        for k in range(640 // C)
    ]
    for d in zd:
        d.wait()
    plsc.subcore_barrier()
    pltpu.sync_copy(row3_hbm.at[w, 0], rowv)
    pltpu.sync_copy(col3_hbm.at[w, 0], colv.at[pl.ds(0, BLK)])
    g = [None] * NCH
    sc = [None] * NCH
    ipf = None
    nsw = 0
    g[0] = pltpu.async_copy(y_hbm.at[colv.at[0]], gbuf.at[0], gsem)
    g[1] = pltpu.async_copy(y_hbm.at[colv.at[1]], gbuf.at[1], gsem)
    for i in range(NCH):
        t = i % BLK
        b = i // BLK
        if t == 0 and b > 0:
            while nsw < i:
                sc[nsw].wait()
                nsw += 1
            pltpu.sync_copy(row3_hbm.at[w, b], rowv)
        if t == 2 and b + 1 < NBLK:
            ipf = pltpu.async_copy(
                col3_hbm.at[w, b + 1],
                colv.at[pl.ds(BLK * ((b + 1) % 2), BLK)], isem)
        while nsw <= i - 1:
            sc[nsw].wait()
            nsw += 1
        if i + 2 < NCH:
            tn = (i + 2) % BLK
            bn = (i + 2) // BLK
            if tn == 0 and bn > 0:
                ipf.wait()
            g[i + 2] = pltpu.async_copy(
                y_hbm.at[colv.at[BLK * (bn % 2) + tn]],
                gbuf.at[(i + 2) % 3], gsem)
        g[i].wait()
        sc[i] = pltpu.async_copy(
            gbuf.at[i % 3], acc_sh.at[rowv.at[t]], ssem, add=True)
    while nsw < NCH:
        sc[nsw].wait()
        nsw += 1
    plsc.subcore_barrier()
    pltpu.sync_copy(acc_sh.at[pl.ds(640 * s, 640)],
                    z2_hbm.at[c, pl.ds(640 * s, 640)])


@functools.partial(
    pl.kernel,
    out_type=jax.ShapeDtypeStruct((NC, NPAD, D), jnp.float32),
    mesh=_mesh,
    scratch_types=[
        pltpu.VMEM_SHARED((NPAD, D), jnp.float32),
        pltpu.VMEM((BLK, C), jnp.int32),
        pltpu.VMEM((2 * BLK, C), jnp.int32),
        pltpu.VMEM((3, C, D), jnp.float32),
        pltpu.SemaphoreType.DMA,
        pltpu.SemaphoreType.DMA,
        pltpu.SemaphoreType.DMA,
        pltpu.SemaphoreType.DMA,
    ],
    compiler_params=_sc_params,
)
def _agg(*args):
    _agg_body(*args)


def _mm_body(z2_ref, dis_ref, w_ref, b_ref, o_ref):
    z = z2_ref[0] + z2_ref[1]
    zd = z * dis_ref[...]
    r = lax.dot_general(
        zd, w_ref[...], (((1,), (1,)), ((), ())),
        preferred_element_type=jnp.float32,
        precision=lax.Precision.HIGHEST,
    )
    o_ref[...] = r + b_ref[...]


def _mm(z2, dis_col, W, b2):
    blk = 256
    grid = NPAD // blk
    return pl.pallas_call(
        _mm_body,
        grid=(grid,),
        in_specs=[
            pl.BlockSpec((NC, blk, D), lambda i: (0, i, 0)),
            pl.BlockSpec((blk, 1), lambda i: (i, 0)),
            pl.BlockSpec((D, D), lambda i: (0, 0)),
            pl.BlockSpec((1, D), lambda i: (0, 0)),
        ],
        out_specs=pl.BlockSpec((blk, D), lambda i: (i, 0)),
        out_shape=jax.ShapeDtypeStruct((N, D), jnp.float32),
    )(z2, dis_col, W, b2)


@jax.jit
def kernel(x, edge_index, W, b):
    row = edge_index[0].astype(jnp.int32)
    col = edge_index[1].astype(jnp.int32)
    rowp = row.reshape(NS, NCHD, C)
    row3 = row.reshape(NW, NBLK, BLK, C)
    col3 = col.reshape(NW, NBLK, BLK, C)
    x_pad = jnp.pad(x, ((0, NPAD - N), (0, 0)))
    dis, y = _prep(rowp, x_pad)
    z2 = _agg(row3, col3, y)
    return _mm(z2, dis.reshape(NPAD, 1), W, b.reshape(1, D))


# speedup vs baseline: 1.2021x; 1.0660x over previous
import functools

import jax
import jax.numpy as jnp
from jax import lax
from jax.experimental import pallas as pl
from jax.experimental.pallas import tpu as pltpu
from jax.experimental.pallas import tpu_sc as plsc

N = 10000
NPAD = 10240
E = 320000
D = 128

NC = 2
NS = 16
NW = NC * NS

C = 80
EPW = E // NW
NCH = EPW // C
EPT = E // NS
NCHD = EPT // C
NPW = NPAD // NW

_mesh = plsc.VectorSubcoreMesh(core_axis_name="c", subcore_axis_name="s")
_sc_params = pltpu.CompilerParams(needs_layout_passes=False)


def _prep_body(rowp_hbm, x_hbm, dis_hbm, y_hbm,
               deg_sh, rowv, onesv, zdeg, dgv, disv, xv, dsem, xsem):
    c = lax.axis_index("c")
    s = lax.axis_index("s")
    for j in range(C // 16):
        onesv[pl.ds(16 * j, 16)] = jnp.ones((16,), jnp.float32)
    for j in range(40):
        zdeg[pl.ds(16 * j, 16)] = jnp.zeros((16,), jnp.float32)
    pltpu.sync_copy(zdeg, deg_sh.at[pl.ds(640 * s, 640)])
    nb = 5120 * c + 320 * s
    xd = pltpu.async_copy(x_hbm.at[pl.ds(nb, NPW)], xv, xsem)
    plsc.subcore_barrier()
    pltpu.sync_copy(rowp_hbm.at[s], rowv)
    descs = [None] * NCHD
    for k in range(NCHD):
        if k >= 25:
            descs[k - 25].wait()
        descs[k] = pltpu.async_copy(onesv, deg_sh.at[rowv.at[k]],
                                    dsem, add=True)
    for k in range(NCHD - 25, NCHD):
        descs[k].wait()
    plsc.subcore_barrier()
    pltpu.sync_copy(deg_sh.at[pl.ds(nb, NPW)], dgv)
    for j in range(NPW // 16):
        dg = dgv[pl.ds(16 * j, 16)]
        dgc = jnp.maximum(dg, 1.0)
        u = lax.bitcast_convert_type(dgc, jnp.int32)
        u = jnp.int32(0x5F3759DF) - (u >> 1)
        h = lax.bitcast_convert_type(u, jnp.float32)
        for _ in range(3):
            h = h * (1.5 - 0.5 * dgc * h * h)
        disv[pl.ds(16 * j, 16)] = jnp.where(dg > 0.0, h, 0.0)
    pltpu.sync_copy(disv, dis_hbm.at[pl.ds(nb, NPW)])
    xd.wait()

    def nbody(n, carry):
        idxn = jnp.zeros((16,), jnp.int32) + n
        s16 = plsc.load_gather(disv, [idxn])
        for j in range(D // 16):
            xv[n, pl.ds(16 * j, 16)] = xv[n, pl.ds(16 * j, 16)] * s16
        return carry

    lax.fori_loop(0, NPW, nbody, 0)
    pltpu.sync_copy(xv, y_hbm.at[pl.ds(nb, NPW)])


@functools.partial(
    pl.kernel,
    out_type=(
        jax.ShapeDtypeStruct((NPAD,), jnp.float32),
        jax.ShapeDtypeStruct((NPAD, D), jnp.float32),
    ),
    mesh=_mesh,
    scratch_types=[
        pltpu.VMEM_SHARED((NPAD,), jnp.float32),
        pltpu.VMEM((NCHD, C), jnp.int32),
        pltpu.VMEM((C,), jnp.float32),
        pltpu.VMEM((640,), jnp.float32),
        pltpu.VMEM((NPW,), jnp.float32),
        pltpu.VMEM((NPW,), jnp.float32),
        pltpu.VMEM((NPW, D), jnp.float32),
        pltpu.SemaphoreType.DMA,
        pltpu.SemaphoreType.DMA,
    ],
    compiler_params=_sc_params,
)
def _prep(*args):
    _prep_body(*args)


BLK = 25
NBLK = NCH // BLK


def _agg_body(row3_hbm, col3_hbm, y_hbm, z2_hbm,
              acc_sh, rowv, colv, gbuf, gsem, ssem, zsem, isem):
    c = lax.axis_index("c")
    s = lax.axis_index("s")
    w = s * NC + c

    def zbody(k, carry):
        for j in range(D // 16):
            gbuf[0, k, pl.ds(16 * j, 16)] = jnp.zeros((16,), jnp.float32)
        return carry

    lax.fori_loop(0, C, zbody, 0)
    zd = [
        pltpu.async_copy(gbuf.at[0], acc_sh.at[pl.ds(640 * s + C * k, C)], zsem)
        for k in range(640 // C)
    ]
    for d in zd:
        d.wait()
    plsc.subcore_barrier()
    pltpu.sync_copy(row3_hbm.at[w, 0], rowv)
    pltpu.sync_copy(col3_hbm.at[w, 0], colv.at[pl.ds(0, BLK)])
    g = [None] * NCH
    sc = [None] * NCH
    ipf = None
    nsw = 0
    g[0] = pltpu.async_copy(y_hbm.at[colv.at[0]], gbuf.at[0], gsem)
    g[1] = pltpu.async_copy(y_hbm.at[colv.at[1]], gbuf.at[1], gsem)
    for i in range(NCH):
        t = i % BLK
        b = i // BLK
        if t == 0 and b > 0:
            while nsw < i:
                sc[nsw].wait()
                nsw += 1
            pltpu.sync_copy(row3_hbm.at[w, b], rowv)
        if t == 2 and b + 1 < NBLK:
            ipf = pltpu.async_copy(
                col3_hbm.at[w, b + 1],
                colv.at[pl.ds(BLK * ((b + 1) % 2), BLK)], isem)
        while nsw <= i - 1:
            sc[nsw].wait()
            nsw += 1
        if i + 2 < NCH:
            tn = (i + 2) % BLK
            bn = (i + 2) // BLK
            if tn == 0 and bn > 0:
                ipf.wait()
            g[i + 2] = pltpu.async_copy(
                y_hbm.at[colv.at[BLK * (bn % 2) + tn]],
                gbuf.at[(i + 2) % 3], gsem)
        g[i].wait()
        sc[i] = pltpu.async_copy(
            gbuf.at[i % 3], acc_sh.at[rowv.at[t]], ssem, add=True)
    while nsw < NCH:
        sc[nsw].wait()
        nsw += 1
    plsc.subcore_barrier()
    pltpu.sync_copy(acc_sh.at[pl.ds(640 * s, 640)],
                    z2_hbm.at[c, pl.ds(640 * s, 640)])


@functools.partial(
    pl.kernel,
    out_type=jax.ShapeDtypeStruct((NC, NPAD, D), jnp.float32),
    mesh=_mesh,
    scratch_types=[
        pltpu.VMEM_SHARED((NPAD, D), jnp.float32),
        pltpu.VMEM((BLK, C), jnp.int32),
        pltpu.VMEM((2 * BLK, C), jnp.int32),
        pltpu.VMEM((3, C, D), jnp.float32),
        pltpu.SemaphoreType.DMA,
        pltpu.SemaphoreType.DMA,
        pltpu.SemaphoreType.DMA,
        pltpu.SemaphoreType.DMA,
    ],
    compiler_params=_sc_params,
)
def _agg(*args):
    _agg_body(*args)


def _mm_body(z2_ref, dis_ref, w_ref, b_ref, o_ref):
    z = z2_ref[0] + z2_ref[1]
    zd = z * dis_ref[...]
    r = lax.dot_general(
        zd, w_ref[...], (((1,), (1,)), ((), ())),
        preferred_element_type=jnp.float32,
        precision=lax.Precision.HIGHEST,
    )
    o_ref[...] = r + b_ref[...]


def _mm(z2, dis_col, W, b2):
    blk = 512
    grid = NPAD // blk
    return pl.pallas_call(
        _mm_body,
        grid=(grid,),
        in_specs=[
            pl.BlockSpec((NC, blk, D), lambda i: (0, i, 0)),
            pl.BlockSpec((blk, 1), lambda i: (i, 0)),
            pl.BlockSpec((D, D), lambda i: (0, 0)),
            pl.BlockSpec((1, D), lambda i: (0, 0)),
        ],
        out_specs=pl.BlockSpec((blk, D), lambda i: (i, 0)),
        out_shape=jax.ShapeDtypeStruct((N, D), jnp.float32),
    )(z2, dis_col, W, b2)


@jax.jit
def kernel(x, edge_index, W, b):
    row = edge_index[0].astype(jnp.int32)
    col = edge_index[1].astype(jnp.int32)
    rowp = row.reshape(NS, NCHD, C)
    row3 = row.reshape(NW, NBLK, BLK, C)
    col3 = col.reshape(NW, NBLK, BLK, C)
    x_pad = jnp.pad(x, ((0, NPAD - N), (0, 0)))
    dis, y = _prep(rowp, x_pad)
    z2 = _agg(row3, col3, y)
    return _mm(z2, dis.reshape(NPAD, 1), W, b.reshape(1, D))


# speedup vs baseline: 1.2418x; 1.0330x over previous
import functools

import jax
import jax.numpy as jnp
from jax import lax
from jax.experimental import pallas as pl
from jax.experimental.pallas import tpu as pltpu
from jax.experimental.pallas import tpu_sc as plsc

N = 10000
NPAD = 10240
E = 320000
D = 128

NC = 2
NS = 16
NW = NC * NS

C = 80
EPW = E // NW
NCH = EPW // C
EPT = E // NS
NCHD = EPT // C
NPW = NPAD // NW

_mesh = plsc.VectorSubcoreMesh(core_axis_name="c", subcore_axis_name="s")
_sc_params = pltpu.CompilerParams(needs_layout_passes=False)


def _prep_body(rowp_hbm, x_hbm, dis_hbm, y_hbm,
               deg_sh, rowv, onesv, zdeg, dgv, disv, xv, dsem, xsem):
    c = lax.axis_index("c")
    s = lax.axis_index("s")
    for j in range(C // 16):
        onesv[pl.ds(16 * j, 16)] = jnp.ones((16,), jnp.float32)
    for j in range(40):
        zdeg[pl.ds(16 * j, 16)] = jnp.zeros((16,), jnp.float32)
    pltpu.sync_copy(zdeg, deg_sh.at[pl.ds(640 * s, 640)])
    nb = 5120 * c + 320 * s
    xd = pltpu.async_copy(x_hbm.at[pl.ds(nb, NPW)], xv, xsem)
    plsc.subcore_barrier()
    pltpu.sync_copy(rowp_hbm.at[s], rowv)
    descs = [None] * NCHD
    for k in range(NCHD):
        if k >= 25:
            descs[k - 25].wait()
        descs[k] = pltpu.async_copy(onesv, deg_sh.at[rowv.at[k]],
                                    dsem, add=True)
    for k in range(NCHD - 25, NCHD):
        descs[k].wait()
    plsc.subcore_barrier()
    pltpu.sync_copy(deg_sh.at[pl.ds(nb, NPW)], dgv)
    for j in range(NPW // 16):
        dg = dgv[pl.ds(16 * j, 16)]
        dgc = jnp.maximum(dg, 1.0)
        u = lax.bitcast_convert_type(dgc, jnp.int32)
        u = jnp.int32(0x5F3759DF) - (u >> 1)
        h = lax.bitcast_convert_type(u, jnp.float32)
        for _ in range(3):
            h = h * (1.5 - 0.5 * dgc * h * h)
        disv[pl.ds(16 * j, 16)] = jnp.where(dg > 0.0, h, 0.0)
    pltpu.sync_copy(disv, dis_hbm.at[pl.ds(nb, NPW)])
    xd.wait()

    def nbody(n, carry):
        idxn = jnp.zeros((16,), jnp.int32) + n
        s16 = plsc.load_gather(disv, [idxn])
        for j in range(D // 16):
            xv[n, pl.ds(16 * j, 16)] = xv[n, pl.ds(16 * j, 16)] * s16
        return carry

    lax.fori_loop(0, NPW, nbody, 0)
    pltpu.sync_copy(xv, y_hbm.at[pl.ds(nb, NPW)])


@functools.partial(
    pl.kernel,
    out_type=(
        jax.ShapeDtypeStruct((NPAD,), jnp.float32),
        jax.ShapeDtypeStruct((NPAD, D), jnp.float32),
    ),
    mesh=_mesh,
    scratch_types=[
        pltpu.VMEM_SHARED((NPAD,), jnp.float32),
        pltpu.VMEM((NCHD, C), jnp.int32),
        pltpu.VMEM((C,), jnp.float32),
        pltpu.VMEM((640,), jnp.float32),
        pltpu.VMEM((NPW,), jnp.float32),
        pltpu.VMEM((NPW,), jnp.float32),
        pltpu.VMEM((NPW, D), jnp.float32),
        pltpu.SemaphoreType.DMA,
        pltpu.SemaphoreType.DMA,
    ],
    compiler_params=_sc_params,
)
def _prep(*args):
    _prep_body(*args)


BLK = 25
NBLK = NCH // BLK


def _agg_body(row3_hbm, col3_hbm, y_hbm, z2_hbm,
              acc_sh, rowv, colv, gbuf, gsem, ssem, zsem, isem):
    c = lax.axis_index("c")
    s = lax.axis_index("s")
    w = s * NC + c

    def zbody(k, carry):
        for j in range(D // 16):
            gbuf[0, k, pl.ds(16 * j, 16)] = jnp.zeros((16,), jnp.float32)
        return carry

    lax.fori_loop(0, C, zbody, 0)
    zd = [
        pltpu.async_copy(gbuf.at[0], acc_sh.at[pl.ds(640 * s + C * k, C)], zsem)
        for k in range(640 // C)
    ]
    for d in zd:
        d.wait()
    plsc.subcore_barrier()
    pltpu.sync_copy(row3_hbm.at[w, 0], rowv)
    pltpu.sync_copy(col3_hbm.at[w, 0], colv.at[pl.ds(0, BLK)])
    g = [None] * NCH
    sc = [None] * NCH
    ipf = None
    nsw = 0
    g[0] = pltpu.async_copy(y_hbm.at[colv.at[0]], gbuf.at[0], gsem)
    g[1] = pltpu.async_copy(y_hbm.at[colv.at[1]], gbuf.at[1], gsem)
    for i in range(NCH):
        t = i % BLK
        b = i // BLK
        if t == 0 and b > 0:
            while nsw < i:
                sc[nsw].wait()
                nsw += 1
            pltpu.sync_copy(row3_hbm.at[w, b], rowv)
        if t == 2 and b + 1 < NBLK:
            ipf = pltpu.async_copy(
                col3_hbm.at[w, b + 1],
                colv.at[pl.ds(BLK * ((b + 1) % 2), BLK)], isem)
        while nsw <= i - 1:
            sc[nsw].wait()
            nsw += 1
        if i + 2 < NCH:
            tn = (i + 2) % BLK
            bn = (i + 2) // BLK
            if tn == 0 and bn > 0:
                ipf.wait()
            g[i + 2] = pltpu.async_copy(
                y_hbm.at[colv.at[BLK * (bn % 2) + tn]],
                gbuf.at[(i + 2) % 3], gsem)
        g[i].wait()
        sc[i] = pltpu.async_copy(
            gbuf.at[i % 3], acc_sh.at[rowv.at[t]], ssem, add=True)
    while nsw < NCH:
        sc[nsw].wait()
        nsw += 1
    plsc.subcore_barrier()
    pltpu.sync_copy(acc_sh.at[pl.ds(640 * s, 640)],
                    z2_hbm.at[c, pl.ds(640 * s, 640)])


@functools.partial(
    pl.kernel,
    out_type=jax.ShapeDtypeStruct((NC, NPAD, D), jnp.float32),
    mesh=_mesh,
    scratch_types=[
        pltpu.VMEM_SHARED((NPAD, D), jnp.float32),
        pltpu.VMEM((BLK, C), jnp.int32),
        pltpu.VMEM((2 * BLK, C), jnp.int32),
        pltpu.VMEM((3, C, D), jnp.float32),
        pltpu.SemaphoreType.DMA,
        pltpu.SemaphoreType.DMA,
        pltpu.SemaphoreType.DMA,
        pltpu.SemaphoreType.DMA,
    ],
    compiler_params=_sc_params,
)
def _agg(*args):
    _agg_body(*args)


def _mm_body(z2_ref, dis_ref, w_ref, b_ref, o_ref):
    z = z2_ref[0] + z2_ref[1]
    zd = z * dis_ref[...]
    r = lax.dot_general(
        zd, w_ref[...], (((1,), (1,)), ((), ())),
        preferred_element_type=jnp.float32,
        precision=lax.Precision.HIGHEST,
    )
    o_ref[...] = r + b_ref[...]


def _mm(z2, dis_col, W, b2):
    blk = 1024
    grid = NPAD // blk
    return pl.pallas_call(
        _mm_body,
        grid=(grid,),
        in_specs=[
            pl.BlockSpec((NC, blk, D), lambda i: (0, i, 0)),
            pl.BlockSpec((blk, 1), lambda i: (i, 0)),
            pl.BlockSpec((D, D), lambda i: (0, 0)),
            pl.BlockSpec((1, D), lambda i: (0, 0)),
        ],
        out_specs=pl.BlockSpec((blk, D), lambda i: (i, 0)),
        out_shape=jax.ShapeDtypeStruct((N, D), jnp.float32),
    )(z2, dis_col, W, b2)


@jax.jit
def kernel(x, edge_index, W, b):
    row = edge_index[0].astype(jnp.int32)
    col = edge_index[1].astype(jnp.int32)
    rowp = row.reshape(NS, NCHD, C)
    row3 = row.reshape(NW, NBLK, BLK, C)
    col3 = col.reshape(NW, NBLK, BLK, C)
    x_pad = jnp.pad(x, ((0, NPAD - N), (0, 0)))
    dis, y = _prep(rowp, x_pad)
    z2 = _agg(row3, col3, y)
    return _mm(z2, dis.reshape(NPAD, 1), W, b.reshape(1, D))


# speedup vs baseline: 1.2537x; 1.0096x over previous
import functools

import jax
import jax.numpy as jnp
from jax import lax
from jax.experimental import pallas as pl
from jax.experimental.pallas import tpu as pltpu
from jax.experimental.pallas import tpu_sc as plsc

N = 10000
NPAD = 10240
E = 320000
D = 128

NC = 2
NS = 16
NW = NC * NS

C = 80
EPW = E // NW
NCH = EPW // C
EPT = E // NS
NCHD = EPT // C
NPW = NPAD // NW

_mesh = plsc.VectorSubcoreMesh(core_axis_name="c", subcore_axis_name="s")
_sc_params = pltpu.CompilerParams(needs_layout_passes=False)


def _prep_body(rowp_hbm, x_hbm, dis_hbm, y_hbm,
               deg_sh, rowv, onesv, zdeg, dgv, disv, xv, dsem, xsem):
    c = lax.axis_index("c")
    s = lax.axis_index("s")
    for j in range(C // 16):
        onesv[pl.ds(16 * j, 16)] = jnp.ones((16,), jnp.float32)
    for j in range(40):
        zdeg[pl.ds(16 * j, 16)] = jnp.zeros((16,), jnp.float32)
    pltpu.sync_copy(zdeg, deg_sh.at[pl.ds(640 * s, 640)])
    nb = 5120 * c + 320 * s
    xd = pltpu.async_copy(x_hbm.at[pl.ds(nb, NPW)], xv, xsem)
    plsc.subcore_barrier()
    pltpu.sync_copy(rowp_hbm.at[s], rowv)
    descs = [None] * NCHD
    for k in range(NCHD):
        if k >= 25:
            descs[k - 25].wait()
        descs[k] = pltpu.async_copy(onesv, deg_sh.at[rowv.at[k]],
                                    dsem, add=True)
    for k in range(NCHD - 25, NCHD):
        descs[k].wait()
    plsc.subcore_barrier()
    pltpu.sync_copy(deg_sh.at[pl.ds(nb, NPW)], dgv)
    for j in range(NPW // 16):
        dg = dgv[pl.ds(16 * j, 16)]
        dgc = jnp.maximum(dg, 1.0)
        u = lax.bitcast_convert_type(dgc, jnp.int32)
        u = jnp.int32(0x5F3759DF) - (u >> 1)
        h = lax.bitcast_convert_type(u, jnp.float32)
        for _ in range(3):
            h = h * (1.5 - 0.5 * dgc * h * h)
        disv[pl.ds(16 * j, 16)] = jnp.where(dg > 0.0, h, 0.0)
    pltpu.sync_copy(disv, dis_hbm.at[pl.ds(nb, NPW)])
    xd.wait()

    def nbody(n, carry):
        idxn = jnp.zeros((16,), jnp.int32) + n
        s16 = plsc.load_gather(disv, [idxn])
        for j in range(D // 16):
            xv[n, pl.ds(16 * j, 16)] = xv[n, pl.ds(16 * j, 16)] * s16
        return carry

    lax.fori_loop(0, NPW, nbody, 0)
    pltpu.sync_copy(xv, y_hbm.at[pl.ds(nb, NPW)])


@functools.partial(
    pl.kernel,
    out_type=(
        jax.ShapeDtypeStruct((NPAD,), jnp.float32),
        jax.ShapeDtypeStruct((NPAD, D), jnp.float32),
    ),
    mesh=_mesh,
    scratch_types=[
        pltpu.VMEM_SHARED((NPAD,), jnp.float32),
        pltpu.VMEM((NCHD, C), jnp.int32),
        pltpu.VMEM((C,), jnp.float32),
        pltpu.VMEM((640,), jnp.float32),
        pltpu.VMEM((NPW,), jnp.float32),
        pltpu.VMEM((NPW,), jnp.float32),
        pltpu.VMEM((NPW, D), jnp.float32),
        pltpu.SemaphoreType.DMA,
        pltpu.SemaphoreType.DMA,
    ],
    compiler_params=_sc_params,
)
def _prep(*args):
    _prep_body(*args)


BLK = 25
NBLK = NCH // BLK


def _agg_body(row3_hbm, col3_hbm, y_hbm, z2_hbm,
              acc_sh, rowv, colv, gbuf, gsem, ssem, zsem, isem):
    c = lax.axis_index("c")
    s = lax.axis_index("s")
    w = s * NC + c

    def zbody(k, carry):
        for j in range(D // 16):
            gbuf[0, k, pl.ds(16 * j, 16)] = jnp.zeros((16,), jnp.float32)
        return carry

    lax.fori_loop(0, C, zbody, 0)
    zd = [
        pltpu.async_copy(gbuf.at[0], acc_sh.at[pl.ds(640 * s + C * k, C)], zsem)
        for k in range(640 // C)
    ]
    for d in zd:
        d.wait()
    plsc.subcore_barrier()
    pltpu.sync_copy(row3_hbm.at[w, 0], rowv)
    pltpu.sync_copy(col3_hbm.at[w, 0], colv.at[pl.ds(0, BLK)])
    g = [None] * NCH
    sc = [None] * NCH
    ipf = None
    nsw = 0
    g[0] = pltpu.async_copy(y_hbm.at[colv.at[0]], gbuf.at[0], gsem)
    g[1] = pltpu.async_copy(y_hbm.at[colv.at[1]], gbuf.at[1], gsem)
    for i in range(NCH):
        t = i % BLK
        b = i // BLK
        if t == 0 and b > 0:
            while nsw < i:
                sc[nsw].wait()
                nsw += 1
            pltpu.sync_copy(row3_hbm.at[w, b], rowv)
        if t == 2 and b + 1 < NBLK:
            ipf = pltpu.async_copy(
                col3_hbm.at[w, b + 1],
                colv.at[pl.ds(BLK * ((b + 1) % 2), BLK)], isem)
        while nsw <= i - 1:
            sc[nsw].wait()
            nsw += 1
        if i + 2 < NCH:
            tn = (i + 2) % BLK
            bn = (i + 2) // BLK
            if tn == 0 and bn > 0:
                ipf.wait()
            g[i + 2] = pltpu.async_copy(
                y_hbm.at[colv.at[BLK * (bn % 2) + tn]],
                gbuf.at[(i + 2) % 3], gsem)
        g[i].wait()
        sc[i] = pltpu.async_copy(
            gbuf.at[i % 3], acc_sh.at[rowv.at[t]], ssem, add=True)
    while nsw < NCH:
        sc[nsw].wait()
        nsw += 1
    plsc.subcore_barrier()
    pltpu.sync_copy(acc_sh.at[pl.ds(640 * s, 640)],
                    z2_hbm.at[c, pl.ds(640 * s, 640)])


@functools.partial(
    pl.kernel,
    out_type=jax.ShapeDtypeStruct((NC, NPAD, D), jnp.float32),
    mesh=_mesh,
    scratch_types=[
        pltpu.VMEM_SHARED((NPAD, D), jnp.float32),
        pltpu.VMEM((BLK, C), jnp.int32),
        pltpu.VMEM((2 * BLK, C), jnp.int32),
        pltpu.VMEM((3, C, D), jnp.float32),
        pltpu.SemaphoreType.DMA,
        pltpu.SemaphoreType.DMA,
        pltpu.SemaphoreType.DMA,
        pltpu.SemaphoreType.DMA,
    ],
    compiler_params=_sc_params,
)
def _agg(*args):
    _agg_body(*args)


def _mm_body(z2_ref, dis_ref, w_ref, b_ref, o_ref):
    z = z2_ref[0] + z2_ref[1]
    zd = z * dis_ref[...]
    r = lax.dot_general(
        zd, w_ref[...], (((1,), (1,)), ((), ())),
        preferred_element_type=jnp.float32,
        precision=lax.Precision.HIGHEST,
    )
    o_ref[...] = r + b_ref[...]


def _mm(z2, dis_col, W, b2):
    blk = 2048
    grid = NPAD // blk
    return pl.pallas_call(
        _mm_body,
        grid=(grid,),
        in_specs=[
            pl.BlockSpec((NC, blk, D), lambda i: (0, i, 0)),
            pl.BlockSpec((blk, 1), lambda i: (i, 0)),
            pl.BlockSpec((D, D), lambda i: (0, 0)),
            pl.BlockSpec((1, D), lambda i: (0, 0)),
        ],
        out_specs=pl.BlockSpec((blk, D), lambda i: (i, 0)),
        out_shape=jax.ShapeDtypeStruct((N, D), jnp.float32),
    )(z2, dis_col, W, b2)


@jax.jit
def kernel(x, edge_index, W, b):
    row = edge_index[0].astype(jnp.int32)
    col = edge_index[1].astype(jnp.int32)
    rowp = row.reshape(NS, NCHD, C)
    row3 = row.reshape(NW, NBLK, BLK, C)
    col3 = col.reshape(NW, NBLK, BLK, C)
    x_pad = jnp.pad(x, ((0, NPAD - N), (0, 0)))
    dis, y = _prep(rowp, x_pad)
    z2 = _agg(row3, col3, y)
    return _mm(z2, dis.reshape(NPAD, 1), W, b.reshape(1, D))


# speedup vs baseline: 1.2695x; 1.0126x over previous
import functools

import jax
import jax.numpy as jnp
from jax import lax
from jax.experimental import pallas as pl
from jax.experimental.pallas import tpu as pltpu
from jax.experimental.pallas import tpu_sc as plsc

N = 10000
NPAD = 10240
E = 320000
D = 128

NC = 2
NS = 16
NW = NC * NS

C = 80
EPW = E // NW
NCH = EPW // C
EPT = E // NS
NCHD = EPT // C
NPW = NPAD // NW

_mesh = plsc.VectorSubcoreMesh(core_axis_name="c", subcore_axis_name="s")
_sc_params = pltpu.CompilerParams(needs_layout_passes=False)


def _prep_body(rowp_hbm, x_hbm, dis_hbm, y_hbm,
               deg_sh, rowv, onesv, zdeg, dgv, disv, xv, dsem, xsem):
    c = lax.axis_index("c")
    s = lax.axis_index("s")
    for j in range(C // 16):
        onesv[pl.ds(16 * j, 16)] = jnp.ones((16,), jnp.float32)
    for j in range(40):
        zdeg[pl.ds(16 * j, 16)] = jnp.zeros((16,), jnp.float32)
    pltpu.sync_copy(zdeg, deg_sh.at[pl.ds(640 * s, 640)])
    nb = 5120 * c + 320 * s
    xd = pltpu.async_copy(x_hbm.at[pl.ds(nb, NPW)], xv, xsem)
    plsc.subcore_barrier()
    pltpu.sync_copy(rowp_hbm.at[s], rowv)
    descs = [None] * NCHD
    for k in range(NCHD):
        if k >= 25:
            descs[k - 25].wait()
        descs[k] = pltpu.async_copy(onesv, deg_sh.at[rowv.at[k]],
                                    dsem, add=True)
    for k in range(NCHD - 25, NCHD):
        descs[k].wait()
    plsc.subcore_barrier()
    pltpu.sync_copy(deg_sh.at[pl.ds(nb, NPW)], dgv)
    for j in range(NPW // 16):
        dg = dgv[pl.ds(16 * j, 16)]
        dgc = jnp.maximum(dg, 1.0)
        u = lax.bitcast_convert_type(dgc, jnp.int32)
        u = jnp.int32(0x5F3759DF) - (u >> 1)
        h = lax.bitcast_convert_type(u, jnp.float32)
        for _ in range(3):
            h = h * (1.5 - 0.5 * dgc * h * h)
        disv[pl.ds(16 * j, 16)] = jnp.where(dg > 0.0, h, 0.0)
    pltpu.sync_copy(disv, dis_hbm.at[pl.ds(nb, NPW)])
    xd.wait()

    def nbody(n, carry):
        idxn = jnp.zeros((16,), jnp.int32) + n
        s16 = plsc.load_gather(disv, [idxn])
        for j in range(D // 16):
            xv[n, pl.ds(16 * j, 16)] = xv[n, pl.ds(16 * j, 16)] * s16
        return carry

    lax.fori_loop(0, NPW, nbody, 0)
    pltpu.sync_copy(xv, y_hbm.at[pl.ds(nb, NPW)])


@functools.partial(
    pl.kernel,
    out_type=(
        jax.ShapeDtypeStruct((NPAD,), jnp.float32),
        jax.ShapeDtypeStruct((NPAD, D), jnp.float32),
    ),
    mesh=_mesh,
    scratch_types=[
        pltpu.VMEM_SHARED((NPAD,), jnp.float32),
        pltpu.VMEM((NCHD, C), jnp.int32),
        pltpu.VMEM((C,), jnp.float32),
        pltpu.VMEM((640,), jnp.float32),
        pltpu.VMEM((NPW,), jnp.float32),
        pltpu.VMEM((NPW,), jnp.float32),
        pltpu.VMEM((NPW, D), jnp.float32),
        pltpu.SemaphoreType.DMA,
        pltpu.SemaphoreType.DMA,
    ],
    compiler_params=_sc_params,
)
def _prep(*args):
    _prep_body(*args)


BLK = 25
NBLK = NCH // BLK


def _agg_body(row3_hbm, col3_hbm, y_hbm, z2_hbm,
              acc_sh, rowv, colv, gbuf, gsem, ssem, zsem, isem):
    c = lax.axis_index("c")
    s = lax.axis_index("s")
    w = s * NC + c

    def zbody(k, carry):
        for j in range(D // 16):
            gbuf[0, k, pl.ds(16 * j, 16)] = jnp.zeros((16,), jnp.float32)
        return carry

    lax.fori_loop(0, C, zbody, 0)
    zd = [
        pltpu.async_copy(gbuf.at[0], acc_sh.at[pl.ds(640 * s + C * k, C)], zsem)
        for k in range(640 // C)
    ]
    for d in zd:
        d.wait()
    plsc.subcore_barrier()
    pltpu.sync_copy(row3_hbm.at[w, 0], rowv)
    pltpu.sync_copy(col3_hbm.at[w, 0], colv.at[pl.ds(0, BLK)])
    g = [None] * NCH
    sc = [None] * NCH
    ipf = None
    nsw = 0
    g[0] = pltpu.async_copy(y_hbm.at[colv.at[0]], gbuf.at[0], gsem)
    g[1] = pltpu.async_copy(y_hbm.at[colv.at[1]], gbuf.at[1], gsem)
    for i in range(NCH):
        t = i % BLK
        b = i // BLK
        if t == 0 and b > 0:
            while nsw < i:
                sc[nsw].wait()
                nsw += 1
            pltpu.sync_copy(row3_hbm.at[w, b], rowv)
        if t == 2 and b + 1 < NBLK:
            ipf = pltpu.async_copy(
                col3_hbm.at[w, b + 1],
                colv.at[pl.ds(BLK * ((b + 1) % 2), BLK)], isem)
        while nsw <= i - 1:
            sc[nsw].wait()
            nsw += 1
        if i + 2 < NCH:
            tn = (i + 2) % BLK
            bn = (i + 2) // BLK
            if tn == 0 and bn > 0:
                ipf.wait()
            g[i + 2] = pltpu.async_copy(
                y_hbm.at[colv.at[BLK * (bn % 2) + tn]],
                gbuf.at[(i + 2) % 3], gsem)
        g[i].wait()
        sc[i] = pltpu.async_copy(
            gbuf.at[i % 3], acc_sh.at[rowv.at[t]], ssem, add=True)
    while nsw < NCH:
        sc[nsw].wait()
        nsw += 1
    plsc.subcore_barrier()
    pltpu.sync_copy(acc_sh.at[pl.ds(640 * s, 640)],
                    z2_hbm.at[c, pl.ds(640 * s, 640)])


@functools.partial(
    pl.kernel,
    out_type=jax.ShapeDtypeStruct((NC, NPAD, D), jnp.float32),
    mesh=_mesh,
    scratch_types=[
        pltpu.VMEM_SHARED((NPAD, D), jnp.float32),
        pltpu.VMEM((BLK, C), jnp.int32),
        pltpu.VMEM((2 * BLK, C), jnp.int32),
        pltpu.VMEM((3, C, D), jnp.float32),
        pltpu.SemaphoreType.DMA,
        pltpu.SemaphoreType.DMA,
        pltpu.SemaphoreType.DMA,
        pltpu.SemaphoreType.DMA,
    ],
    compiler_params=_sc_params,
)
def _agg(*args):
    _agg_body(*args)


def _mm_body(z2_ref, dis_ref, w_ref, b_ref, o_ref):
    z = z2_ref[0] + z2_ref[1]
    zd = z * dis_ref[...]
    r = lax.dot_general(
        zd, w_ref[...], (((1,), (1,)), ((), ())),
        preferred_element_type=jnp.float32,
    )
    o_ref[...] = r + b_ref[...]


def _mm(z2, dis_col, W, b2):
    blk = 2048
    grid = NPAD // blk
    return pl.pallas_call(
        _mm_body,
        grid=(grid,),
        in_specs=[
            pl.BlockSpec((NC, blk, D), lambda i: (0, i, 0)),
            pl.BlockSpec((blk, 1), lambda i: (i, 0)),
            pl.BlockSpec((D, D), lambda i: (0, 0)),
            pl.BlockSpec((1, D), lambda i: (0, 0)),
        ],
        out_specs=pl.BlockSpec((blk, D), lambda i: (i, 0)),
        out_shape=jax.ShapeDtypeStruct((N, D), jnp.float32),
    )(z2, dis_col, W, b2)


@jax.jit
def kernel(x, edge_index, W, b):
    row = edge_index[0].astype(jnp.int32)
    col = edge_index[1].astype(jnp.int32)
    rowp = row.reshape(NS, NCHD, C)
    row3 = row.reshape(NW, NBLK, BLK, C)
    col3 = col.reshape(NW, NBLK, BLK, C)
    x_pad = jnp.pad(x, ((0, NPAD - N), (0, 0)))
    dis, y = _prep(rowp, x_pad)
    z2 = _agg(row3, col3, y)
    return _mm(z2, dis.reshape(NPAD, 1), W, b.reshape(1, D))
